# Initial kernel scaffold; baseline (speedup 1.0000x reference)
import functools

import jax
import jax.numpy as jnp
from jax import lax
from jax.experimental import pallas as pl
from jax.experimental.pallas import tpu as pltpu
from jax.experimental.pallas import tpu_sc as plsc

B = 16384
L = 50
D = 64
OUT = 64

NC = 2
NS = 16
NW = NC * NS
BAGS_W = B // NW
CH = 4
ROWS = CH * L
NCHUNK = BAGS_W // CH
IDX_W = BAGS_W * L


def _sc_bag_sums(x_flat, table):
    mesh = plsc.VectorSubcoreMesh(core_axis_name="c", subcore_axis_name="s")

    @functools.partial(
        pl.kernel,
        out_type=jax.ShapeDtypeStruct((B, D), jnp.float32),
        mesh=mesh,
        scratch_types=[
            pltpu.VMEM((IDX_W,), jnp.int32),
            pltpu.VMEM((ROWS, D), jnp.float32),
            pltpu.VMEM((ROWS, D), jnp.float32),
            pltpu.VMEM((BAGS_W, D), jnp.float32),
            pltpu.SemaphoreType.DMA,
            pltpu.SemaphoreType.DMA,
        ],
    )
    def k(x_hbm, tab_hbm, out_hbm, idx_v, g0, g1, acc_v, s0, s1):
        wid = lax.axis_index("s") * NC + lax.axis_index("c")
        pltpu.sync_copy(x_hbm.at[pl.ds(wid * IDX_W, IDX_W)], idx_v)

        def fire(c, buf, sem):
            r = c * ROWS
            pltpu.async_copy(
                tab_hbm.at[idx_v.at[pl.ds(r, 128)]], buf.at[pl.ds(0, 128)], sem)
            pltpu.async_copy(
                tab_hbm.at[idx_v.at[pl.ds(r + 128, ROWS - 128)]],
                buf.at[pl.ds(128, ROWS - 128)], sem)

        def drain(buf, sem):
            pltpu.make_async_copy(tab_hbm.at[pl.ds(0, ROWS)], buf, sem).wait()

        def reduce_chunk(buf, c):
            ob = c * CH

            def bag_body(bb, carry):
                r0 = bb * L
                for q in range(D // 16):
                    col = pl.ds(q * 16, 16)
                    a = buf[r0, col]
                    for j in range(1, L):
                        a = a + buf[r0 + j, col]
                    acc_v[ob + bb, col] = a
                return carry

            lax.fori_loop(0, CH, bag_body, 0)

        fire(0, g0, s0)

        def step(s, carry):
            c0 = 2 * s
            fire(c0 + 1, g1, s1)
            drain(g0, s0)
            reduce_chunk(g0, c0)

            @pl.when(c0 + 2 < NCHUNK)
            def _():
                fire(c0 + 2, g0, s0)

            drain(g1, s1)
            reduce_chunk(g1, c0 + 1)
            return carry

        lax.fori_loop(0, NCHUNK // 2, step, 0)
        pltpu.sync_copy(acc_v, out_hbm.at[pl.ds(wid * BAGS_W, BAGS_W)])

    return k(x_flat, table)


def _tc_head(bag_sums, Wt, b2):
    BLK = 1024

    def body(p_ref, w_ref, b_ref, o_ref):
        y = jnp.dot(p_ref[...], w_ref[...],
                    preferred_element_type=jnp.float32) + b_ref[...]
        m = jnp.max(y, axis=1, keepdims=True)
        e = jnp.exp(y - m)
        o_ref[...] = e / jnp.sum(e, axis=1, keepdims=True)

    return pl.pallas_call(
        body,
        grid=(B // BLK,),
        in_specs=[
            pl.BlockSpec((BLK, D), lambda i: (i, 0)),
            pl.BlockSpec((D, OUT), lambda i: (0, 0)),
            pl.BlockSpec((1, OUT), lambda i: (0, 0)),
        ],
        out_specs=pl.BlockSpec((BLK, OUT), lambda i: (i, 0)),
        out_shape=jax.ShapeDtypeStruct((B, OUT), jnp.float32),
    )(bag_sums, Wt, b2)


def kernel(x, emb_table, W, b):
    x_flat = x.reshape(B * L).astype(jnp.int32)
    sums = _sc_bag_sums(x_flat, emb_table)
    Wt = W.T.astype(jnp.float32) * jnp.float32(1.0 / L)
    return _tc_head(sums, Wt, b[None, :].astype(jnp.float32))


# speedup vs baseline: 2.3511x; 2.3511x over previous
import functools

import jax
import jax.numpy as jnp
from jax import lax
from jax.experimental import pallas as pl
from jax.experimental.pallas import tpu as pltpu
from jax.experimental.pallas import tpu_sc as plsc

B = 16384
L = 50
D = 64
OUT = 64

NC = 2
NS = 16
NW = NC * NS
BAGS_W = B // NW
CH = 4
ROWS = CH * L
NCHUNK = BAGS_W // CH
IDX_W = BAGS_W * L


def _sc_bag_sums(x_flat, table):
    mesh = plsc.VectorSubcoreMesh(core_axis_name="c", subcore_axis_name="s")

    @functools.partial(
        pl.kernel,
        out_type=jax.ShapeDtypeStruct((B, D), jnp.float32),
        mesh=mesh,
        compiler_params=pltpu.CompilerParams(use_tc_tiling_on_sc=False),
        scratch_types=[
            pltpu.VMEM((IDX_W,), jnp.int32),
            pltpu.VMEM((ROWS, D), jnp.float32),
            pltpu.VMEM((ROWS, D), jnp.float32),
            pltpu.VMEM((BAGS_W, D), jnp.float32),
            pltpu.SemaphoreType.DMA,
            pltpu.SemaphoreType.DMA,
        ],
    )
    def k(x_hbm, tab_hbm, out_hbm, idx_v, g0, g1, acc_v, s0, s1):
        wid = lax.axis_index("s") * NC + lax.axis_index("c")
        pltpu.sync_copy(x_hbm.at[pl.ds(wid * IDX_W, IDX_W)], idx_v)

        def fire(c, buf, sem):
            r = c * ROWS
            pltpu.async_copy(
                tab_hbm.at[idx_v.at[pl.ds(r, 128)]], buf.at[pl.ds(0, 128)], sem)
            pltpu.async_copy(
                tab_hbm.at[idx_v.at[pl.ds(r + 128, ROWS - 128)]],
                buf.at[pl.ds(128, ROWS - 128)], sem)

        def drain(buf, sem):
            pltpu.make_async_copy(tab_hbm.at[pl.ds(0, ROWS)], buf, sem).wait()

        def reduce_chunk(buf, c):
            ob = c * CH

            def bag_body(bb, carry):
                r0 = bb * L
                for q in range(D // 16):
                    col = pl.ds(q * 16, 16)
                    a = buf[r0, col]
                    for j in range(1, L):
                        a = a + buf[r0 + j, col]
                    acc_v[ob + bb, col] = a
                return carry

            lax.fori_loop(0, CH, bag_body, 0)

        fire(0, g0, s0)

        def step(s, carry):
            c0 = 2 * s
            fire(c0 + 1, g1, s1)
            drain(g0, s0)
            reduce_chunk(g0, c0)

            @pl.when(c0 + 2 < NCHUNK)
            def _():
                fire(c0 + 2, g0, s0)

            drain(g1, s1)
            reduce_chunk(g1, c0 + 1)
            return carry

        lax.fori_loop(0, NCHUNK // 2, step, 0)
        pltpu.sync_copy(acc_v, out_hbm.at[pl.ds(wid * BAGS_W, BAGS_W)])

    return k(x_flat, table)


def _tc_head(bag_sums, Wt, b2):
    BLK = 1024

    def body(p_ref, w_ref, b_ref, o_ref):
        y = jnp.dot(p_ref[...], w_ref[...],
                    preferred_element_type=jnp.float32) + b_ref[...]
        m = jnp.max(y, axis=1, keepdims=True)
        e = jnp.exp(y - m)
        o_ref[...] = e / jnp.sum(e, axis=1, keepdims=True)

    return pl.pallas_call(
        body,
        grid=(B // BLK,),
        in_specs=[
            pl.BlockSpec((BLK, D), lambda i: (i, 0)),
            pl.BlockSpec((D, OUT), lambda i: (0, 0)),
            pl.BlockSpec((1, OUT), lambda i: (0, 0)),
        ],
        out_specs=pl.BlockSpec((BLK, OUT), lambda i: (i, 0)),
        out_shape=jax.ShapeDtypeStruct((B, OUT), jnp.float32),
    )(bag_sums, Wt, b2)


def kernel(x, emb_table, W, b):
    x_flat = x.reshape(B * L).astype(jnp.int32)
    sums = _sc_bag_sums(x_flat, emb_table)
    Wt = W.T.astype(jnp.float32) * jnp.float32(1.0 / L)
    return _tc_head(sums, Wt, b[None, :].astype(jnp.float32))
